# baseline (speedup 1.0000x reference)
import jax
import jax.numpy as jnp
from jax import lax
from jax.experimental import pallas as pl
from jax.experimental.pallas import tpu as pltpu
from jax.experimental.pallas import tpu_sc as plsc

B = 16384
F = 26
NN = 13
VOCAB = 100000
D = 32
NT = F + NN

NC = 2
NS = 16
NW = NC * NS

NROW = NT * D
RPW = NROW // NW
CL = 4096
NCH = B // CL
VPC = CL // 16


def _sc_tokenizer(tbl_hbm, xc_hbm, xn_hbm, w_hbm, b_hbm, out_hbm,
                  row_v, idx_v, xn_v, wb_v, o_v, osem):
    cid = lax.axis_index("c")
    sid = lax.axis_index("s")
    wid = sid * NC + cid
    r0 = wid * RPW

    pltpu.sync_copy(w_hbm, wb_v.at[pl.ds(0, D)])
    pltpu.sync_copy(b_hbm, wb_v.at[pl.ds(D, D)])

    zero = jnp.float32(0.0)
    one = jnp.float32(1.0)

    def drain_o(slot):
        pltpu.make_async_copy(o_v.at[slot], out_hbm.at[0, 0, pl.ds(0, CL)],
                              osem.at[slot]).wait()

    def row_body(j, prev_f):
        r = r0 + j
        t = r // D
        d = lax.rem(r, D)
        is_cat = t < F

        @pl.when(is_cat & (t != prev_f))
        def _():
            pltpu.sync_copy(xc_hbm.at[t], idx_v)

        @pl.when(is_cat)
        def _():
            pltpu.sync_copy(tbl_hbm.at[t, d], row_v)

            for c in range(NCH):
                slot = c & 1
                if c >= 2:
                    drain_o(slot)
                else:
                    @pl.when(j > 0)
                    def _():
                        drain_o(slot)

                def vbody(v, _):
                    p = c * CL + v * 16
                    iv = idx_v[pl.ds(p, 16)]
                    m = iv != 0
                    g = plsc.load_gather(row_v, [iv], mask=m)
                    o_v[slot, pl.ds(v * 16, 16)] = g
                    return 0

                lax.fori_loop(0, VPC, vbody, 0, unroll=8)
                pltpu.async_copy(o_v.at[slot],
                                 out_hbm.at[t, d, pl.ds(c * CL, CL)],
                                 osem.at[slot])

        @pl.when(jnp.logical_not(is_cat))
        def _():
            i = t - F
            dsplat = jnp.full((16,), d, jnp.int32)
            wd = plsc.load_gather(wb_v, [dsplat])
            bd = plsc.load_gather(wb_v, [dsplat + D])

            for c in range(NCH):
                slot = c & 1
                if c >= 2:
                    drain_o(slot)
                else:
                    @pl.when(j > 0)
                    def _():
                        drain_o(slot)
                pltpu.sync_copy(xn_hbm.at[i, pl.ds(c * CL, CL)], xn_v)

                def vbody(v, _):
                    xv = xn_v[pl.ds(v * 16, 16)]
                    o_v[slot, pl.ds(v * 16, 16)] = xv * wd + bd
                    return 0

                lax.fori_loop(0, VPC, vbody, 0, unroll=8)
                pltpu.async_copy(o_v.at[slot],
                                 out_hbm.at[t, d, pl.ds(c * CL, CL)],
                                 osem.at[slot])

        return jnp.where(is_cat, t, prev_f)

    lax.fori_loop(0, RPW, row_body, jnp.int32(-1))
    drain_o(0)
    drain_o(1)


@jax.jit
def _run(t3, xc_t, xn_t, w_flat, b_vec):
    mesh = plsc.VectorSubcoreMesh(core_axis_name="c", subcore_axis_name="s")
    fn = pl.kernel(
        _sc_tokenizer,
        out_type=jax.ShapeDtypeStruct((NT, D, B), jnp.float32),
        mesh=mesh,
        scratch_types=[
            pltpu.VMEM((VOCAB,), jnp.float32),
            pltpu.VMEM((B,), jnp.int32),
            pltpu.VMEM((CL,), jnp.float32),
            pltpu.VMEM((2 * D,), jnp.float32),
            pltpu.VMEM((2, CL), jnp.float32),
            pltpu.SemaphoreType.DMA((2,)),
        ],
        compiler_params=pltpu.CompilerParams(needs_layout_passes=False),
    )
    return fn(t3, xc_t, xn_t, w_flat, b_vec)


def kernel(x_cat, x_num, tables, w, b):
    t3 = tables.transpose(0, 2, 1)
    xc_t = x_cat.astype(jnp.int32).T
    xn_t = x_num.T
    out = _run(t3, xc_t, xn_t, w[:, 0], b)
    return out.transpose(2, 0, 1)


# speedup vs baseline: 1.0479x; 1.0479x over previous
import jax
import jax.numpy as jnp
from jax import lax
from jax.experimental import pallas as pl
from jax.experimental.pallas import tpu as pltpu
from jax.experimental.pallas import tpu_sc as plsc

B = 16384
F = 26
NN = 13
VOCAB = 100000
D = 32
NT = F + NN

NC = 2
NS = 16
NW = NC * NS

NROW = NT * D
RPW = NROW // NW
CL = 4096
NCH = B // CL
VPC = CL // 16


def _sc_tokenizer(tbl_hbm, xc_hbm, xn_hbm, w_hbm, b_hbm, out_hbm,
                  row_v, idx_v, xn_v, wb_v, o_v, osem):
    cid = lax.axis_index("c")
    sid = lax.axis_index("s")
    wid = sid * NC + cid
    r0 = wid * RPW

    pltpu.sync_copy(w_hbm, wb_v.at[pl.ds(0, D)])
    pltpu.sync_copy(b_hbm, wb_v.at[pl.ds(D, D)])

    zero = jnp.float32(0.0)
    one = jnp.float32(1.0)

    def drain_o(slot):
        pltpu.make_async_copy(o_v.at[slot], out_hbm.at[0, 0, pl.ds(0, CL)],
                              osem.at[slot]).wait()

    def row_body(j, prev_f):
        r = r0 + j
        t = r // D
        d = lax.rem(r, D)
        is_cat = t < F

        @pl.when(is_cat & (t != prev_f))
        def _():
            pltpu.sync_copy(xc_hbm.at[t], idx_v)

        @pl.when(is_cat)
        def _():
            pltpu.sync_copy(tbl_hbm.at[t, d], row_v)

            for c in range(NCH):
                slot = c & 1
                if c >= 2:
                    drain_o(slot)

                def vbody(v, _):
                    p = c * CL + v * 16
                    iv = idx_v[pl.ds(p, 16)]
                    g = plsc.load_gather(row_v, [iv])
                    m = jnp.where(iv == 0, zero, one)
                    o_v[slot, pl.ds(v * 16, 16)] = g * m
                    return 0

                lax.fori_loop(0, VPC, vbody, 0, unroll=8)
                pltpu.async_copy(o_v.at[slot],
                                 out_hbm.at[t, d, pl.ds(c * CL, CL)],
                                 osem.at[slot])
            drain_o(0)
            drain_o(1)

        @pl.when(jnp.logical_not(is_cat))
        def _():
            i = t - F
            dsplat = jnp.full((16,), d, jnp.int32)
            wd = plsc.load_gather(wb_v, [dsplat])
            bd = plsc.load_gather(wb_v, [dsplat + D])

            for c in range(NCH):
                slot = c & 1
                if c >= 2:
                    drain_o(slot)
                pltpu.sync_copy(xn_hbm.at[i, pl.ds(c * CL, CL)], xn_v)

                def vbody(v, _):
                    xv = xn_v[pl.ds(v * 16, 16)]
                    o_v[slot, pl.ds(v * 16, 16)] = xv * wd + bd
                    return 0

                lax.fori_loop(0, VPC, vbody, 0, unroll=8)
                pltpu.async_copy(o_v.at[slot],
                                 out_hbm.at[t, d, pl.ds(c * CL, CL)],
                                 osem.at[slot])
            drain_o(0)
            drain_o(1)

        return jnp.where(is_cat, t, prev_f)

    lax.fori_loop(0, RPW, row_body, jnp.int32(-1))


@jax.jit
def _run(t3, xc_t, xn_t, w_flat, b_vec):
    mesh = plsc.VectorSubcoreMesh(core_axis_name="c", subcore_axis_name="s")
    fn = pl.kernel(
        _sc_tokenizer,
        out_type=jax.ShapeDtypeStruct((NT, D, B), jnp.float32),
        mesh=mesh,
        scratch_types=[
            pltpu.VMEM((VOCAB,), jnp.float32),
            pltpu.VMEM((B,), jnp.int32),
            pltpu.VMEM((CL,), jnp.float32),
            pltpu.VMEM((2 * D,), jnp.float32),
            pltpu.VMEM((2, CL), jnp.float32),
            pltpu.SemaphoreType.DMA((2,)),
        ],
        compiler_params=pltpu.CompilerParams(needs_layout_passes=False),
    )
    return fn(t3, xc_t, xn_t, w_flat, b_vec)


def kernel(x_cat, x_num, tables, w, b):
    t3 = tables.transpose(0, 2, 1)
    xc_t = x_cat.astype(jnp.int32).T
    xn_t = x_num.T
    out = _run(t3, xc_t, xn_t, w[:, 0], b)
    return out.transpose(2, 0, 1)


# speedup vs baseline: 1.0698x; 1.0209x over previous
import jax
import jax.numpy as jnp
from jax import lax
from jax.experimental import pallas as pl
from jax.experimental.pallas import tpu as pltpu
from jax.experimental.pallas import tpu_sc as plsc

B = 16384
F = 26
NN = 13
VOCAB = 100000
D = 32
NT = F + NN

NC = 2
NS = 16
NW = NC * NS

NROW = NT * D
RPW = NROW // NW
CL = 2048
NCH = B // CL
NSLOT = 4
VPC = CL // 16


def _sc_tokenizer(tbl_hbm, xc_hbm, xn_hbm, w_hbm, b_hbm, out_hbm,
                  row_v, idx_v, xn_v, wb_v, o_v, osem):
    cid = lax.axis_index("c")
    sid = lax.axis_index("s")
    wid = sid * NC + cid
    r0 = wid * RPW

    pltpu.sync_copy(w_hbm, wb_v.at[pl.ds(0, D)])
    pltpu.sync_copy(b_hbm, wb_v.at[pl.ds(D, D)])

    zero = jnp.float32(0.0)
    one = jnp.float32(1.0)

    def drain_o(slot):
        pltpu.make_async_copy(o_v.at[slot], out_hbm.at[0, 0, pl.ds(0, CL)],
                              osem.at[slot]).wait()

    def row_body(j, prev_f):
        r = r0 + j
        t = r // D
        d = lax.rem(r, D)
        is_cat = t < F

        @pl.when(is_cat & (t != prev_f))
        def _():
            pltpu.sync_copy(xc_hbm.at[t], idx_v)

        @pl.when(is_cat)
        def _():
            pltpu.sync_copy(tbl_hbm.at[t, d], row_v)

            for c in range(NCH):
                slot = c % NSLOT
                if c >= NSLOT:
                    drain_o(slot)

                def vbody(v, _):
                    p = c * CL + v * 16
                    iv = idx_v[pl.ds(p, 16)]
                    g = plsc.load_gather(row_v, [iv])
                    m = jnp.where(iv == 0, zero, one)
                    o_v[slot, pl.ds(v * 16, 16)] = g * m
                    return 0

                lax.fori_loop(0, VPC, vbody, 0, unroll=8)
                pltpu.async_copy(o_v.at[slot],
                                 out_hbm.at[t, d, pl.ds(c * CL, CL)],
                                 osem.at[slot])
            for s_ in range(NSLOT):
                drain_o(s_)

        @pl.when(jnp.logical_not(is_cat))
        def _():
            i = t - F
            dsplat = jnp.full((16,), d, jnp.int32)
            wd = plsc.load_gather(wb_v, [dsplat])
            bd = plsc.load_gather(wb_v, [dsplat + D])

            for c in range(NCH):
                slot = c % NSLOT
                if c >= NSLOT:
                    drain_o(slot)
                pltpu.sync_copy(xn_hbm.at[i, pl.ds(c * CL, CL)], xn_v)

                def vbody(v, _):
                    xv = xn_v[pl.ds(v * 16, 16)]
                    o_v[slot, pl.ds(v * 16, 16)] = xv * wd + bd
                    return 0

                lax.fori_loop(0, VPC, vbody, 0, unroll=8)
                pltpu.async_copy(o_v.at[slot],
                                 out_hbm.at[t, d, pl.ds(c * CL, CL)],
                                 osem.at[slot])
            for s_ in range(NSLOT):
                drain_o(s_)

        return jnp.where(is_cat, t, prev_f)

    lax.fori_loop(0, RPW, row_body, jnp.int32(-1))


@jax.jit
def _run(t3, xc_t, xn_t, w_flat, b_vec):
    mesh = plsc.VectorSubcoreMesh(core_axis_name="c", subcore_axis_name="s")
    fn = pl.kernel(
        _sc_tokenizer,
        out_type=jax.ShapeDtypeStruct((NT, D, B), jnp.float32),
        mesh=mesh,
        scratch_types=[
            pltpu.VMEM((VOCAB,), jnp.float32),
            pltpu.VMEM((B,), jnp.int32),
            pltpu.VMEM((CL,), jnp.float32),
            pltpu.VMEM((2 * D,), jnp.float32),
            pltpu.VMEM((NSLOT, CL), jnp.float32),
            pltpu.SemaphoreType.DMA((NSLOT,)),
        ],
        compiler_params=pltpu.CompilerParams(needs_layout_passes=False),
    )
    return fn(t3, xc_t, xn_t, w_flat, b_vec)


def kernel(x_cat, x_num, tables, w, b):
    t3 = tables.transpose(0, 2, 1)
    xc_t = x_cat.astype(jnp.int32).T
    xn_t = x_num.T
    out = _run(t3, xc_t, xn_t, w[:, 0], b)
    return out.transpose(2, 0, 1)


# speedup vs baseline: 1.0802x; 1.0098x over previous
import jax
import jax.numpy as jnp
from jax import lax
from jax.experimental import pallas as pl
from jax.experimental.pallas import tpu as pltpu
from jax.experimental.pallas import tpu_sc as plsc

B = 16384
F = 26
NN = 13
VOCAB = 100000
D = 32
NT = F + NN

NC = 2
NS = 16
NW = NC * NS

NROW = NT * D
RPW = NROW // NW
CL = 2048
NCH = B // CL
NSLOT = 4
VPC = CL // 16


def _sc_tokenizer(tbl_hbm, xc_hbm, xn_hbm, w_hbm, b_hbm, out_hbm,
                  row_v, idx_v, xn_v, wb_v, o_v, osem):
    cid = lax.axis_index("c")
    sid = lax.axis_index("s")
    wid = sid * NC + cid
    r0 = wid * RPW

    pltpu.sync_copy(w_hbm, wb_v.at[pl.ds(0, D)])
    pltpu.sync_copy(b_hbm, wb_v.at[pl.ds(D, D)])

    zero = jnp.float32(0.0)
    one = jnp.float32(1.0)

    def drain_o(slot):
        pltpu.make_async_copy(o_v.at[slot], out_hbm.at[0, 0, pl.ds(0, CL)],
                              osem.at[slot]).wait()

    def row_body(j, prev_f):
        r = r0 + j
        t = r // D
        d = lax.rem(r, D)
        is_cat = t < F

        @pl.when(is_cat & (t != prev_f))
        def _():
            pltpu.sync_copy(xc_hbm.at[t], idx_v)

        @pl.when(is_cat)
        def _():
            pltpu.sync_copy(tbl_hbm.at[t, d], row_v)

            for c in range(NCH):
                slot = c % NSLOT
                if c >= NSLOT:
                    drain_o(slot)
                else:
                    @pl.when(j > 0)
                    def _():
                        drain_o(slot)

                def vbody(v, _):
                    p = c * CL + v * 16
                    iv = idx_v[pl.ds(p, 16)]
                    g = plsc.load_gather(row_v, [iv])
                    m = jnp.where(iv == 0, zero, one)
                    o_v[slot, pl.ds(v * 16, 16)] = g * m
                    return 0

                lax.fori_loop(0, VPC, vbody, 0, unroll=8)
                pltpu.async_copy(o_v.at[slot],
                                 out_hbm.at[t, d, pl.ds(c * CL, CL)],
                                 osem.at[slot])

        @pl.when(jnp.logical_not(is_cat))
        def _():
            i = t - F
            dsplat = jnp.full((16,), d, jnp.int32)
            wd = plsc.load_gather(wb_v, [dsplat])
            bd = plsc.load_gather(wb_v, [dsplat + D])

            for c in range(NCH):
                slot = c % NSLOT
                if c >= NSLOT:
                    drain_o(slot)
                else:
                    @pl.when(j > 0)
                    def _():
                        drain_o(slot)
                pltpu.sync_copy(xn_hbm.at[i, pl.ds(c * CL, CL)], xn_v)

                def vbody(v, _):
                    xv = xn_v[pl.ds(v * 16, 16)]
                    o_v[slot, pl.ds(v * 16, 16)] = xv * wd + bd
                    return 0

                lax.fori_loop(0, VPC, vbody, 0, unroll=8)
                pltpu.async_copy(o_v.at[slot],
                                 out_hbm.at[t, d, pl.ds(c * CL, CL)],
                                 osem.at[slot])

        return jnp.where(is_cat, t, prev_f)

    lax.fori_loop(0, RPW, row_body, jnp.int32(-1))
    for s_ in range(NSLOT):
        drain_o(s_)


@jax.jit
def _run(t3, xc_t, xn_t, w_flat, b_vec):
    mesh = plsc.VectorSubcoreMesh(core_axis_name="c", subcore_axis_name="s")
    fn = pl.kernel(
        _sc_tokenizer,
        out_type=jax.ShapeDtypeStruct((NT, D, B), jnp.float32),
        mesh=mesh,
        scratch_types=[
            pltpu.VMEM((VOCAB,), jnp.float32),
            pltpu.VMEM((B,), jnp.int32),
            pltpu.VMEM((CL,), jnp.float32),
            pltpu.VMEM((2 * D,), jnp.float32),
            pltpu.VMEM((NSLOT, CL), jnp.float32),
            pltpu.SemaphoreType.DMA((NSLOT,)),
        ],
        compiler_params=pltpu.CompilerParams(needs_layout_passes=False),
    )
    return fn(t3, xc_t, xn_t, w_flat, b_vec)


def kernel(x_cat, x_num, tables, w, b):
    t3 = tables.transpose(0, 2, 1)
    xc_t = x_cat.astype(jnp.int32).T
    xn_t = x_num.T
    out = _run(t3, xc_t, xn_t, w[:, 0], b)
    return out.transpose(2, 0, 1)
